# Initial kernel scaffold; baseline (speedup 1.0000x reference)
import jax
import jax.numpy as jnp
from jax.experimental import pallas as pl


def kernel(hidden_states, all_indices, W_seq, hidden_embeddings, cp_weight):
    raise NotImplementedError("write your pallas kernel here")


# speedup vs baseline: 5.7754x; 5.7754x over previous
import functools

import jax
import jax.numpy as jnp
from jax import lax
from jax.experimental import pallas as pl
from jax.experimental.pallas import tpu as pltpu
from jax.experimental.pallas import tpu_sc as plsc


def _tc_body(idx_ref, hs_ref, wseq_ref, he_ref, cp_ref, g_ref, f_ref):
    H = hs_ref.shape[1]
    x = idx_ref[...].astype(jnp.float32)
    p = lax.broadcasted_iota(jnp.int32, (128, 64), 0)
    q = lax.broadcasted_iota(jnp.int32, (128, 64), 1)
    w = jnp.where(p == 2 * q, float(H), jnp.where(p == 2 * q + 1, 1.0, 0.0))
    f = lax.dot_general(x, w, (((1,), (0,)), ((), ())),
                        preferred_element_type=jnp.float32,
                        precision=lax.Precision.HIGHEST)
    f_ref[...] = (f + 0.5).astype(jnp.int32)

    @pl.when(pl.program_id(0) == 0)
    def _():
        a = lax.dot_general(hs_ref[...], wseq_ref[...], (((1,), (1,)), ((), ())),
                            preferred_element_type=jnp.float32)
        b = he_ref[...] * cp_ref[...]
        g_ref[...] = lax.dot_general(a, b, (((1,), (1,)), ((), ())),
                                     preferred_element_type=jnp.float32)


def _tc_stage(idx2d, hs2, wseq, he, cp, bm):
    nrows = idx2d.shape[0]
    S, H = hs2.shape
    grid = (nrows // bm,)
    return pl.pallas_call(
        _tc_body,
        grid=grid,
        in_specs=[
            pl.BlockSpec((bm, 128), lambda i: (i, 0)),
            pl.BlockSpec((S, H), lambda i: (0, 0)),
            pl.BlockSpec(wseq.shape, lambda i: (0, 0)),
            pl.BlockSpec(he.shape, lambda i: (0, 0)),
            pl.BlockSpec(cp.shape, lambda i: (0, 0)),
        ],
        out_specs=[
            pl.BlockSpec((S, H), lambda i: (0, 0)),
            pl.BlockSpec((bm, 64), lambda i: (i, 0)),
        ],
        out_shape=[
            jax.ShapeDtypeStruct((S, H), jnp.float32),
            jax.ShapeDtypeStruct((nrows, 64), jnp.int32),
        ],
    )(idx2d, hs2, wseq, he, cp)


def _sc_gather(f_flat, g_flat, n_total):
    NW = 32
    n_per_tile = n_total // NW
    CH = 2048
    KD = CH // 128
    n_chunks = n_per_tile // CH
    mesh = plsc.VectorSubcoreMesh(core_axis_name="c", subcore_axis_name="s")

    @functools.partial(
        pl.kernel,
        mesh=mesh,
        out_type=jax.ShapeDtypeStruct((n_total,), jnp.float32),
        scratch_types=[
            pltpu.VMEM((CH,), jnp.int32),
            pltpu.VMEM((CH,), jnp.float32),
            pltpu.SemaphoreType.DMA,
        ],
    )
    def sc_kernel(f_hbm, g_hbm, out_hbm, fbuf, rbuf, sem):
        wid = lax.axis_index("s") * 2 + lax.axis_index("c")
        base = wid * n_per_tile

        def chunk_body(k, carry):
            off = base + k * CH
            pltpu.sync_copy(f_hbm.at[pl.ds(off, CH)], fbuf)
            copies = [
                pltpu.async_copy(
                    g_hbm.at[fbuf.at[pl.ds(j * 128, 128)]],
                    rbuf.at[pl.ds(j * 128, 128)],
                    sem,
                )
                for j in range(KD)
            ]
            for c in copies:
                c.wait()
            pltpu.sync_copy(rbuf, out_hbm.at[pl.ds(off, CH)])
            return carry

        lax.fori_loop(0, n_chunks, chunk_body, 0)

    return sc_kernel(f_flat, g_flat)


def kernel(hidden_states, all_indices, W_seq, hidden_embeddings, cp_weight):
    B, S, H = hidden_states.shape
    N = all_indices.shape[0]
    hs2 = hidden_states.reshape(S, H)
    idx2d = all_indices.reshape(N // 64, 128)
    g, f = _tc_stage(idx2d, hs2, W_seq, hidden_embeddings, cp_weight, bm=2048)
    out = _sc_gather(f.reshape(N), g.reshape(S * H), N)
    return out.reshape(B, S, H)


# speedup vs baseline: 71.0635x; 12.3045x over previous
import functools

import jax
import jax.numpy as jnp
from jax import lax
from jax.experimental import pallas as pl
from jax.experimental.pallas import tpu as pltpu
from jax.experimental.pallas import tpu_sc as plsc


def _g_body(hs_ref, wseq_ref, he_ref, cp_ref, g_ref):
    a = lax.dot_general(hs_ref[...], wseq_ref[...], (((1,), (1,)), ((), ())),
                        preferred_element_type=jnp.float32)
    b = he_ref[...] * cp_ref[...]
    g_ref[...] = lax.dot_general(a, b, (((1,), (1,)), ((), ())),
                                 preferred_element_type=jnp.float32)


def _tc_stage(hs2, wseq, he, cp):
    S, H = hs2.shape
    return pl.pallas_call(
        _g_body,
        out_shape=jax.ShapeDtypeStruct((S, H), jnp.float32),
    )(hs2, wseq, he, cp)


def _sc_gather(seq, hid, g_flat, n_total, H):
    NW = 32
    n_per_tile = n_total // NW
    CH = 2048
    KD = CH // 128
    n_chunks = n_per_tile // CH
    mesh = plsc.VectorSubcoreMesh(core_axis_name="c", subcore_axis_name="s")

    @functools.partial(
        pl.kernel,
        mesh=mesh,
        out_type=jax.ShapeDtypeStruct((n_total,), jnp.float32),
        scratch_types=[
            pltpu.VMEM((CH,), jnp.int32),
            pltpu.VMEM((CH,), jnp.int32),
            pltpu.VMEM((CH,), jnp.int32),
            pltpu.VMEM((CH,), jnp.float32),
            pltpu.SemaphoreType.DMA,
        ],
    )
    def sc_kernel(seq_hbm, hid_hbm, g_hbm, out_hbm, sbuf, hbuf, fbuf, rbuf, sem):
        wid = lax.axis_index("s") * 2 + lax.axis_index("c")
        base = wid * n_per_tile

        def chunk_body(k, carry):
            off = base + k * CH
            pltpu.sync_copy(seq_hbm.at[pl.ds(off, CH)], sbuf)
            pltpu.sync_copy(hid_hbm.at[pl.ds(off, CH)], hbuf)

            def flat_body(j, c2):
                sl = pl.ds(j * 16, 16)
                fbuf[sl] = sbuf[sl] * H + hbuf[sl]
                return c2

            lax.fori_loop(0, CH // 16, flat_body, 0)
            copies = [
                pltpu.async_copy(
                    g_hbm.at[fbuf.at[pl.ds(j * 128, 128)]],
                    rbuf.at[pl.ds(j * 128, 128)],
                    sem,
                )
                for j in range(KD)
            ]
            for c in copies:
                c.wait()
            pltpu.sync_copy(rbuf, out_hbm.at[pl.ds(off, CH)])
            return carry

        lax.fori_loop(0, n_chunks, chunk_body, 0)

    return sc_kernel(seq, hid, g_flat)


def kernel(hidden_states, all_indices, W_seq, hidden_embeddings, cp_weight):
    B, S, H = hidden_states.shape
    N = all_indices.shape[0]
    hs2 = hidden_states.reshape(S, H)
    g = _tc_stage(hs2, W_seq, hidden_embeddings, cp_weight)
    seq = all_indices[:, 0]
    hid = all_indices[:, 1]
    out = _sc_gather(seq, hid, g.reshape(S * H), N, H)
    return out.reshape(B, S, H)


# speedup vs baseline: 104.3591x; 1.4685x over previous
import functools

import jax
import jax.numpy as jnp
from jax import lax
from jax.experimental import pallas as pl
from jax.experimental.pallas import tpu as pltpu
from jax.experimental.pallas import tpu_sc as plsc


def _g_body(hs_ref, wseq_ref, he_ref, cp_ref, l_ref, a_ref):
    @pl.when(pl.program_id(0) == 0)
    def _():
        a_ref[...] = lax.dot_general(hs_ref[...], wseq_ref[...],
                                     (((1,), (1,)), ((), ())),
                                     preferred_element_type=jnp.float32)
    b = he_ref[...] * cp_ref[...]
    l_ref[...] = lax.dot_general(a_ref[...], b, (((1,), (1,)), ((), ())),
                                 preferred_element_type=jnp.float32)


def _tc_stage(hs2, wseq, he, cp):
    S, H = hs2.shape
    R = wseq.shape[0]
    nstrip = H // 128
    return pl.pallas_call(
        _g_body,
        grid=(nstrip,),
        in_specs=[
            pl.BlockSpec((S, H), lambda k: (0, 0)),
            pl.BlockSpec(wseq.shape, lambda k: (0, 0)),
            pl.BlockSpec((128, R), lambda k: (k, 0)),
            pl.BlockSpec(cp.shape, lambda k: (0, 0)),
        ],
        out_specs=pl.BlockSpec((S, 128), lambda k: (k, 0)),
        out_shape=jax.ShapeDtypeStruct((nstrip * S, 128), jnp.float32),
        scratch_shapes=[pltpu.VMEM((S, R), jnp.float32)],
    )(hs2, wseq, he, cp)


def _sc_gather(seq, hid, l_flat, n_total):
    NW = 32
    n_per_tile = n_total // NW
    CH = 4096
    KD = CH // 128
    n_chunks = n_per_tile // CH
    mesh = plsc.VectorSubcoreMesh(core_axis_name="c", subcore_axis_name="s")

    @functools.partial(
        pl.kernel,
        mesh=mesh,
        out_type=jax.ShapeDtypeStruct((n_total,), jnp.float32),
        scratch_types=[
            pltpu.VMEM((CH,), jnp.int32),
            pltpu.VMEM((CH,), jnp.int32),
            pltpu.VMEM((CH,), jnp.int32),
            pltpu.VMEM((CH,), jnp.int32),
            pltpu.VMEM((CH,), jnp.int32),
            pltpu.VMEM((CH,), jnp.int32),
            pltpu.VMEM((CH,), jnp.float32),
            pltpu.VMEM((CH,), jnp.float32),
            pltpu.SemaphoreType.DMA,
            pltpu.SemaphoreType.DMA,
        ],
    )
    def sc_kernel(seq_hbm, hid_hbm, l_hbm, out_hbm,
                  sbuf0, sbuf1, hbuf0, hbuf1, fbuf0, fbuf1, rbuf0, rbuf1,
                  sem0, sem1):
        sbuf = (sbuf0, sbuf1)
        hbuf = (hbuf0, hbuf1)
        fbuf = (fbuf0, fbuf1)
        rbuf = (rbuf0, rbuf1)
        sems = (sem0, sem1)
        wid = lax.axis_index("s") * 2 + lax.axis_index("c")
        base = wid * n_per_tile

        def load_flat_fire(k):
            p = k % 2
            off = base + k * CH
            pltpu.sync_copy(seq_hbm.at[pl.ds(off, CH)], sbuf[p])
            pltpu.sync_copy(hid_hbm.at[pl.ds(off, CH)], hbuf[p])

            def flat_body(j, c2):
                sl = pl.ds(j * 16, 16)
                s = sbuf[p][sl]
                h = hbuf[p][sl]
                fbuf[p][sl] = (
                    lax.shift_left(lax.shift_right_logical(h, 7), 18)
                    | lax.shift_left(s, 7)
                    | (h & 127)
                )
                return c2

            lax.fori_loop(0, CH // 16, flat_body, 0)
            return [
                pltpu.async_copy(
                    l_hbm.at[fbuf[p].at[pl.ds(j * 128, 128)]],
                    rbuf[p].at[pl.ds(j * 128, 128)],
                    sems[p],
                )
                for j in range(KD)
            ]

        pending = load_flat_fire(0)
        for k in range(n_chunks):
            nxt = load_flat_fire(k + 1) if k + 1 < n_chunks else None
            for c in pending:
                c.wait()
            pltpu.sync_copy(rbuf[k % 2], out_hbm.at[pl.ds(base + k * CH, CH)])
            pending = nxt

    return sc_kernel(seq, hid, l_flat)


def kernel(hidden_states, all_indices, W_seq, hidden_embeddings, cp_weight):
    B, S, H = hidden_states.shape
    N = all_indices.shape[0]
    hs2 = hidden_states.reshape(S, H)
    l_tab = _tc_stage(hs2, W_seq, hidden_embeddings, cp_weight)
    seq = all_indices[:, 0]
    hid = all_indices[:, 1]
    out = _sc_gather(seq, hid, l_tab.reshape(S * H), N)
    return out.reshape(B, S, H)
